# Initial kernel scaffold; baseline (speedup 1.0000x reference)
import jax
import jax.numpy as jnp
from jax.experimental import pallas as pl


def kernel(indices, eps, mean_tables, logvar_tables):
    raise NotImplementedError("write your pallas kernel here")


# speedup vs baseline: 1.1953x; 1.1953x over previous
import jax
import jax.numpy as jnp
from jax import lax
from jax.experimental import pallas as pl
from jax.experimental.pallas import tpu as pltpu
from jax.experimental.pallas import tpu_sc as plsc

N_FIELDS = 26
VOCAB = 100000
D = 32
B = 16384

ROWS = N_FIELDS * B
NW = 32
RPW = ROWS // NW
C = 128
NCHUNK = RPW // C


def _sc_body(idx_hbm, eps_hbm, mean_hbm, lv_hbm,
             means_out, lvs_out, zs_out,
             idx_v, m_v, l_v, e_v, in_sem, out_sem):
    wid = lax.axis_index("s") * 2 + lax.axis_index("c")
    base = wid * RPW
    pltpu.sync_copy(idx_hbm.at[pl.ds(base, RPW)], idx_v)

    def chunk(c, carry):
        rb = base + c * C
        isl = idx_v.at[pl.ds(c * C, C)]
        g1 = pltpu.async_copy(mean_hbm.at[isl], m_v, in_sem)
        g2 = pltpu.async_copy(lv_hbm.at[isl], l_v, in_sem)
        g3 = pltpu.async_copy(eps_hbm.at[pl.ds(rb, C)], e_v, in_sem)
        g1.wait()
        g2.wait()
        g3.wait()

        def row(i, carry2):
            for j in range(D // 16):
                sl = pl.ds(j * 16, 16)
                e_v[i, sl] = m_v[i, sl] + jnp.exp(l_v[i, sl] * 0.5) * e_v[i, sl]
            return carry2

        lax.fori_loop(0, C, row, 0)

        o1 = pltpu.async_copy(m_v, means_out.at[pl.ds(rb, C)], out_sem)
        o2 = pltpu.async_copy(l_v, lvs_out.at[pl.ds(rb, C)], out_sem)
        o3 = pltpu.async_copy(e_v, zs_out.at[pl.ds(rb, C)], out_sem)
        o1.wait()
        o2.wait()
        o3.wait()
        return carry

    lax.fori_loop(0, NCHUNK, chunk, 0)


def kernel(indices, eps, mean_tables, logvar_tables):
    offs = (jnp.arange(N_FIELDS, dtype=jnp.int32) * VOCAB)[:, None]
    gidx = (indices + offs).reshape(ROWS)
    eps_f = eps.reshape(ROWS, D)
    mean_f = mean_tables.reshape(N_FIELDS * VOCAB, D)
    lv_f = logvar_tables.reshape(N_FIELDS * VOCAB, D)

    out_t = (
        jax.ShapeDtypeStruct((ROWS, D), jnp.float32),
        jax.ShapeDtypeStruct((ROWS, D), jnp.float32),
        jax.ShapeDtypeStruct((ROWS, D), jnp.float32),
    )
    f = pl.kernel(
        _sc_body,
        out_type=out_t,
        mesh=plsc.VectorSubcoreMesh(core_axis_name="c", subcore_axis_name="s"),
        compiler_params=pltpu.CompilerParams(use_tc_tiling_on_sc=False),
        scratch_types=[
            pltpu.VMEM((RPW,), jnp.int32),
            pltpu.VMEM((C, D), jnp.float32),
            pltpu.VMEM((C, D), jnp.float32),
            pltpu.VMEM((C, D), jnp.float32),
            pltpu.SemaphoreType.DMA,
            pltpu.SemaphoreType.DMA,
        ],
    )
    means, lvs, zs = f(gidx, eps_f, mean_f, lv_f)
    shape = (N_FIELDS, B, D)
    return means.reshape(shape), lvs.reshape(shape), zs.reshape(shape)


# speedup vs baseline: 1.2594x; 1.0536x over previous
import jax
import jax.numpy as jnp
from jax import lax
from jax.experimental import pallas as pl
from jax.experimental.pallas import tpu as pltpu
from jax.experimental.pallas import tpu_sc as plsc

N_FIELDS = 26
VOCAB = 100000
D = 32
B = 16384

ROWS = N_FIELDS * B
NW = 32
RPW = ROWS // NW
C = 128
NCHUNK = RPW // C
NBUF = 4
NOUTER = NCHUNK // NBUF


def _sc_body(idx_hbm, eps_hbm, mean_hbm, lv_hbm,
             means_out, lvs_out, zs_out,
             idx_v, m_in, l_in, e_in, m_st, l_st, z_st,
             in_sem, out_sem):
    wid = lax.axis_index("s") * 2 + lax.axis_index("c")
    base = wid * RPW
    pltpu.sync_copy(idx_hbm.at[pl.ds(base, RPW)], idx_v)

    def start_in(c, b):
        rb = base + c * C
        isl = idx_v.at[pl.ds(c * C, C)]
        pltpu.async_copy(mean_hbm.at[isl], m_in.at[b], in_sem.at[b])
        pltpu.async_copy(lv_hbm.at[isl], l_in.at[b], in_sem.at[b])
        pltpu.async_copy(eps_hbm.at[pl.ds(rb, C)], e_in.at[b], in_sem.at[b])

    def wait_in(b):
        for dst in (m_in.at[b], l_in.at[b], e_in.at[b]):
            pltpu.make_async_copy(eps_hbm.at[pl.ds(base, C)], dst, in_sem.at[b]).wait()

    def start_out(c, b):
        rb = base + c * C
        pltpu.async_copy(m_st.at[b], means_out.at[pl.ds(rb, C)], out_sem.at[b])
        pltpu.async_copy(l_st.at[b], lvs_out.at[pl.ds(rb, C)], out_sem.at[b])
        pltpu.async_copy(z_st.at[b], zs_out.at[pl.ds(rb, C)], out_sem.at[b])

    def wait_out(b):
        for src in (m_st.at[b], l_st.at[b], z_st.at[b]):
            pltpu.make_async_copy(src, means_out.at[pl.ds(base, C)], out_sem.at[b]).wait()

    def compute(b):
        mb, lb, eb = m_in.at[b], l_in.at[b], e_in.at[b]
        mo, lo, zo = m_st.at[b], l_st.at[b], z_st.at[b]

        def row(i, carry):
            r = i * 4
            for k in range(4):
                for j in range(D // 16):
                    sl = pl.ds(j * 16, 16)
                    m = mb[r + k, sl]
                    lv = lb[r + k, sl]
                    e = eb[r + k, sl]
                    mo[r + k, sl] = m
                    lo[r + k, sl] = lv
                    zo[r + k, sl] = m + jnp.exp(lv * 0.5) * e
            return carry

        lax.fori_loop(0, C // 4, row, 0)

    for b in range(NBUF):
        start_in(b, b)

    def outer(g, carry):
        for b in range(NBUF):
            c = g * NBUF + b
            wait_in(b)

            @pl.when(g > 0)
            def _():
                wait_out(b)

            compute(b)
            start_out(c, b)

            @pl.when(g < NOUTER - 1)
            def _():
                start_in(c + NBUF, b)

        return carry

    lax.fori_loop(0, NOUTER, outer, 0)

    for b in range(NBUF):
        wait_out(b)


def kernel(indices, eps, mean_tables, logvar_tables):
    offs = (jnp.arange(N_FIELDS, dtype=jnp.int32) * VOCAB)[:, None]
    gidx = (indices + offs).reshape(ROWS)
    eps_f = eps.reshape(ROWS, D)
    mean_f = mean_tables.reshape(N_FIELDS * VOCAB, D)
    lv_f = logvar_tables.reshape(N_FIELDS * VOCAB, D)

    out_t = (
        jax.ShapeDtypeStruct((ROWS, D), jnp.float32),
        jax.ShapeDtypeStruct((ROWS, D), jnp.float32),
        jax.ShapeDtypeStruct((ROWS, D), jnp.float32),
    )
    f = pl.kernel(
        _sc_body,
        out_type=out_t,
        mesh=plsc.VectorSubcoreMesh(core_axis_name="c", subcore_axis_name="s"),
        compiler_params=pltpu.CompilerParams(use_tc_tiling_on_sc=False),
        scratch_types=[
            pltpu.VMEM((RPW,), jnp.int32),
            pltpu.VMEM((NBUF, C, D), jnp.float32),
            pltpu.VMEM((NBUF, C, D), jnp.float32),
            pltpu.VMEM((NBUF, C, D), jnp.float32),
            pltpu.VMEM((NBUF, C, D), jnp.float32),
            pltpu.VMEM((NBUF, C, D), jnp.float32),
            pltpu.VMEM((NBUF, C, D), jnp.float32),
            pltpu.SemaphoreType.DMA((NBUF,)),
            pltpu.SemaphoreType.DMA((NBUF,)),
        ],
    )
    means, lvs, zs = f(gidx, eps_f, mean_f, lv_f)
    shape = (N_FIELDS, B, D)
    return means.reshape(shape), lvs.reshape(shape), zs.reshape(shape)


# speedup vs baseline: 1.2600x; 1.0005x over previous
import jax
import jax.numpy as jnp
from jax import lax
from jax.experimental import pallas as pl
from jax.experimental.pallas import tpu as pltpu
from jax.experimental.pallas import tpu_sc as plsc

N_FIELDS = 26
VOCAB = 100000
D = 32
B = 16384

ROWS = N_FIELDS * B
NW = 32
RPW = ROWS // NW
C = 128
CPB = B // C
NCHUNK = RPW // C
NBUF = 4
NOUTER = NCHUNK // NBUF


def _sc_body(idx_hbm, eps_hbm, mean_hbm, lv_hbm,
             means_out, lvs_out, zs_out,
             idx_v, m_in, l_in, e_in, m_st, l_st, z_st,
             in_sem, out_sem):
    wid = lax.axis_index("s") * 2 + lax.axis_index("c")
    base = wid * RPW
    pltpu.sync_copy(idx_hbm.at[pl.ds(base, RPW)], idx_v)

    def field_off(c):
        q = wid * NCHUNK + c
        f = q // CPB
        bo = (q % CPB) * C
        return f, bo

    def start_in(c, b):
        f, bo = field_off(c)
        isl = idx_v.at[pl.ds(c * C, C)]
        pltpu.async_copy(mean_hbm.at[f].at[isl], m_in.at[b], in_sem.at[b])
        pltpu.async_copy(lv_hbm.at[f].at[isl], l_in.at[b], in_sem.at[b])
        pltpu.async_copy(eps_hbm.at[f, pl.ds(bo, C)], e_in.at[b], in_sem.at[b])

    def wait_in(b):
        for dst in (m_in.at[b], l_in.at[b], e_in.at[b]):
            pltpu.make_async_copy(eps_hbm.at[0, pl.ds(0, C)], dst, in_sem.at[b]).wait()

    def start_out(c, b):
        f, bo = field_off(c)
        pltpu.async_copy(m_st.at[b], means_out.at[f, pl.ds(bo, C)], out_sem.at[b])
        pltpu.async_copy(l_st.at[b], lvs_out.at[f, pl.ds(bo, C)], out_sem.at[b])
        pltpu.async_copy(z_st.at[b], zs_out.at[f, pl.ds(bo, C)], out_sem.at[b])

    def wait_out(b):
        for src in (m_st.at[b], l_st.at[b], z_st.at[b]):
            pltpu.make_async_copy(src, means_out.at[0, pl.ds(0, C)], out_sem.at[b]).wait()

    def compute(b):
        mb, lb, eb = m_in.at[b], l_in.at[b], e_in.at[b]
        mo, lo, zo = m_st.at[b], l_st.at[b], z_st.at[b]

        def row(i, carry):
            r = i * 4
            for k in range(4):
                for j in range(D // 16):
                    sl = pl.ds(j * 16, 16)
                    m = mb[r + k, sl]
                    lv = lb[r + k, sl]
                    e = eb[r + k, sl]
                    mo[r + k, sl] = m
                    lo[r + k, sl] = lv
                    zo[r + k, sl] = m + jnp.exp(lv * 0.5) * e
            return carry

        lax.fori_loop(0, C // 4, row, 0)

    for b in range(NBUF):
        start_in(b, b)

    def outer(g, carry):
        for b in range(NBUF):
            c = g * NBUF + b
            wait_in(b)

            @pl.when(g > 0)
            def _():
                wait_out(b)

            compute(b)
            start_out(c, b)

            @pl.when(g < NOUTER - 1)
            def _():
                start_in(c + NBUF, b)

        return carry

    lax.fori_loop(0, NOUTER, outer, 0)

    for b in range(NBUF):
        wait_out(b)


def kernel(indices, eps, mean_tables, logvar_tables):
    gidx = indices.reshape(ROWS)

    out_t = (
        jax.ShapeDtypeStruct((N_FIELDS, B, D), jnp.float32),
        jax.ShapeDtypeStruct((N_FIELDS, B, D), jnp.float32),
        jax.ShapeDtypeStruct((N_FIELDS, B, D), jnp.float32),
    )
    f = pl.kernel(
        _sc_body,
        out_type=out_t,
        mesh=plsc.VectorSubcoreMesh(core_axis_name="c", subcore_axis_name="s"),
        compiler_params=pltpu.CompilerParams(use_tc_tiling_on_sc=False),
        scratch_types=[
            pltpu.VMEM((RPW,), jnp.int32),
            pltpu.VMEM((NBUF, C, D), jnp.float32),
            pltpu.VMEM((NBUF, C, D), jnp.float32),
            pltpu.VMEM((NBUF, C, D), jnp.float32),
            pltpu.VMEM((NBUF, C, D), jnp.float32),
            pltpu.VMEM((NBUF, C, D), jnp.float32),
            pltpu.VMEM((NBUF, C, D), jnp.float32),
            pltpu.SemaphoreType.DMA((NBUF,)),
            pltpu.SemaphoreType.DMA((NBUF,)),
        ],
    )
    return f(gidx, eps, mean_tables, logvar_tables)
